# Initial kernel scaffold; baseline (speedup 1.0000x reference)
import jax
import jax.numpy as jnp
from jax.experimental import pallas as pl


def kernel(x, edge_index, edge_weight, p, W_ih, W_hh, b_ih, b_hh, W_init, W_lin, b_lin):
    raise NotImplementedError("write your pallas kernel here")


# speedup vs baseline: 11.3208x; 11.3208x over previous
import functools

import jax
import jax.numpy as jnp
from jax import lax
from jax.experimental import pallas as pl
from jax.experimental.pallas import tpu as pltpu
from jax.experimental.pallas import tpu_sc as plsc

N = 10000
E = 320000
C = 128

_NC = 2
_NS = 16
_NW = _NC * _NS
_EPW = E // _NW
_K = 80
_NCHUNK = _EPW // _K
_NP = 10240
_RPT = _NP // _NS

_mesh = plsc.VectorSubcoreMesh(core_axis_name="c", subcore_axis_name="s")


def _score_body(x_ref, p_ref, o_ref):
    pv = p_ref[...]
    inv = lax.rsqrt(jnp.sum(pv * pv))
    z = jnp.sum(x_ref[...] * pv, axis=1, keepdims=True)
    o_ref[...] = jnp.tanh(z * inv)


def _scores(x, p2):
    return pl.pallas_call(
        _score_body,
        grid=(10,),
        in_specs=[
            pl.BlockSpec((N // 10, C), lambda i: (i, 0)),
            pl.BlockSpec((1, C), lambda i: (0, 0)),
        ],
        out_specs=pl.BlockSpec((N // 10, 1), lambda i: (i, 0)),
        out_shape=jax.ShapeDtypeStruct((N, 1), jnp.float32),
    )(x, p2)


def _gru_body(xt_ref, wi_ref, wh_ref, bi_ref, bh_ref, h0_ref, w_ref):
    xt = xt_ref[...]
    h0 = h0_ref[...]
    dn = (((1,), (1,)), ((), ()))
    gi = lax.dot_general(xt, wi_ref[...], dn,
                         preferred_element_type=jnp.float32) + bi_ref[...]
    gh = lax.dot_general(h0, wh_ref[...], dn,
                         preferred_element_type=jnp.float32) + bh_ref[...]
    r = jax.nn.sigmoid(gi[:, :C] + gh[:, :C])
    z = jax.nn.sigmoid(gi[:, C:2 * C] + gh[:, C:2 * C])
    nc = jnp.tanh(gi[:, 2 * C:] + r * gh[:, 2 * C:])
    w_ref[...] = (1.0 - z) * nc + z * h0


def _gru(x_tilde, W_ih, W_hh, bi2, bh2, W_init):
    return pl.pallas_call(
        _gru_body,
        out_shape=jax.ShapeDtypeStruct((C, C), jnp.float32),
    )(x_tilde, W_ih, W_hh, bi2, bh2, W_init)


@functools.partial(
    pl.kernel,
    out_type=jax.ShapeDtypeStruct((_NC, _NP, 16), jnp.float32),
    mesh=_mesh,
    scratch_types=[
        pltpu.VMEM_SHARED((_NP, 16), jnp.float32),
        pltpu.VMEM((_K,), jnp.int32),
        pltpu.VMEM((_K,), jnp.float32),
        pltpu.VMEM((_K, 16), jnp.float32),
        pltpu.VMEM((_RPT, 16), jnp.float32),
    ],
)
def _deg_kernel(dst_hbm, w_hbm, out_hbm, dacc, didx, wv, wrow, zbuf):
    c = lax.axis_index("c")
    s = lax.axis_index("s")
    wid = c * _NS + s

    def _zrow(i, _):
        zbuf[i, :] = jnp.zeros((16,), jnp.float32)
        return 0
    lax.fori_loop(0, _RPT, _zrow, 0)
    pltpu.sync_copy(zbuf, dacc.at[pl.ds(s * _RPT, _RPT), :])
    plsc.subcore_barrier()

    base = wid * _EPW

    def _chunk(it, _):
        off = base + it * _K
        pltpu.sync_copy(dst_hbm.at[pl.ds(off, _K)], didx)
        pltpu.sync_copy(w_hbm.at[pl.ds(off, _K)], wv)

        def _grp(g, __):
            wg = wv[pl.ds(g * 16, 16)]
            for l in range(16):
                wrow[g * 16 + l, :] = jnp.broadcast_to(wg[l], (16,))
            return 0
        lax.fori_loop(0, _K // 16, _grp, 0)
        pltpu.sync_copy(wrow, dacc.at[didx], add=True)
        return 0
    lax.fori_loop(0, _NCHUNK, _chunk, 0)
    plsc.subcore_barrier()

    @pl.when(s == 0)
    def _():
        pltpu.sync_copy(dacc, out_hbm.at[c])


def _y_body(x_ref, w_ref, dp_ref, y_ref, dis_ref):
    d = jnp.sum(dp_ref[...], axis=1, keepdims=True) + 1.0
    dis = lax.rsqrt(d)
    xw = lax.dot_general(x_ref[...], w_ref[...], (((1,), (0,)), ((), ())),
                         preferred_element_type=jnp.float32)
    y_ref[...] = xw * dis
    dis_ref[...] = dis


def _y_dis(x, W, dp):
    return pl.pallas_call(
        _y_body,
        grid=(10,),
        in_specs=[
            pl.BlockSpec((N // 10, C), lambda i: (i, 0)),
            pl.BlockSpec((C, C), lambda i: (0, 0)),
            pl.BlockSpec((N // 10, _NC), lambda i: (i, 0)),
        ],
        out_specs=[
            pl.BlockSpec((N // 10, C), lambda i: (i, 0)),
            pl.BlockSpec((N // 10, 1), lambda i: (i, 0)),
        ],
        out_shape=[
            jax.ShapeDtypeStruct((N, C), jnp.float32),
            jax.ShapeDtypeStruct((N, 1), jnp.float32),
        ],
    )(x, W, dp)


@functools.partial(
    pl.kernel,
    out_type=jax.ShapeDtypeStruct((_NC, _NP, C), jnp.float32),
    mesh=_mesh,
    scratch_types=[
        pltpu.VMEM_SHARED((_NP, C), jnp.float32),
        pltpu.VMEM((_K,), jnp.int32),
        pltpu.VMEM((_K,), jnp.int32),
        pltpu.VMEM((_K,), jnp.float32),
        pltpu.VMEM((_K, C), jnp.float32),
        pltpu.VMEM((_RPT // 5, C), jnp.float32),
        pltpu.SemaphoreType.DMA,
    ],
)
def _edge_kernel(src_hbm, dst_hbm, w_hbm, y_hbm, out_hbm,
                 acc, sidx, didx, wv, rows, zbuf, sem):
    c = lax.axis_index("c")
    s = lax.axis_index("s")
    wid = c * _NS + s

    def _zrow(i, _):
        for ci in range(C // 16):
            zbuf[i, pl.ds(ci * 16, 16)] = jnp.zeros((16,), jnp.float32)
        return 0
    lax.fori_loop(0, _RPT // 5, _zrow, 0)
    for j in range(5):
        pltpu.sync_copy(zbuf, acc.at[pl.ds(s * _RPT + j * (_RPT // 5),
                                           _RPT // 5), :])
    plsc.subcore_barrier()

    base = wid * _EPW

    def _chunk(it, _):
        off = base + it * _K
        pltpu.sync_copy(src_hbm.at[pl.ds(off, _K)], sidx)
        pltpu.sync_copy(dst_hbm.at[pl.ds(off, _K)], didx)
        pltpu.sync_copy(w_hbm.at[pl.ds(off, _K)], wv)
        pltpu.async_copy(y_hbm.at[sidx], rows, sem).wait()

        def _grp(g, __):
            wg = wv[pl.ds(g * 16, 16)]
            for l in range(16):
                wj = wg[l]
                j = g * 16 + l
                for ci in range(C // 16):
                    sl = pl.ds(ci * 16, 16)
                    rows[j, sl] = rows[j, sl] * wj
            return 0
        lax.fori_loop(0, _K // 16, _grp, 0)
        pltpu.sync_copy(rows, acc.at[didx], add=True)
        return 0
    lax.fori_loop(0, _NCHUNK, _chunk, 0)
    plsc.subcore_barrier()
    pltpu.sync_copy(acc.at[pl.ds(s * _RPT, _RPT), :],
                    out_hbm.at[c, pl.ds(s * _RPT, _RPT), :])


def _fin_body(a0_ref, a1_ref, y_ref, dis_ref, wl_ref, bl_ref, o_ref):
    t = (a0_ref[...] + a1_ref[...] + y_ref[...]) * dis_ref[...]
    h = jnp.maximum(t, 0.0)
    o_ref[...] = lax.dot_general(h, wl_ref[...], (((1,), (0,)), ((), ())),
                                 preferred_element_type=jnp.float32) + bl_ref[...]


def _final(a0, a1, y, dis, W_lin, bl2):
    return pl.pallas_call(
        _fin_body,
        grid=(10,),
        in_specs=[
            pl.BlockSpec((N // 10, C), lambda i: (i, 0)),
            pl.BlockSpec((N // 10, C), lambda i: (i, 0)),
            pl.BlockSpec((N // 10, C), lambda i: (i, 0)),
            pl.BlockSpec((N // 10, 1), lambda i: (i, 0)),
            pl.BlockSpec((C, 1), lambda i: (0, 0)),
            pl.BlockSpec((1, 1), lambda i: (0, 0)),
        ],
        out_specs=pl.BlockSpec((N // 10, 1), lambda i: (i, 0)),
        out_shape=jax.ShapeDtypeStruct((N, 1), jnp.float32),
    )(a0, a1, y, dis, W_lin, bl2)


def kernel(x, edge_index, edge_weight, p, W_ih, W_hh, b_ih, b_hh,
           W_init, W_lin, b_lin):
    src32 = edge_index[0].astype(jnp.int32)
    dst32 = edge_index[1].astype(jnp.int32)
    ew = edge_weight.astype(jnp.float32)

    score = _scores(x, p.reshape(1, C))[:, 0]
    topv, perm = lax.top_k(score, C)
    x_tilde = x[perm] * topv[:, None]

    W = _gru(x_tilde, W_ih, W_hh, b_ih.reshape(1, 3 * C),
             b_hh.reshape(1, 3 * C), W_init)

    degp = _deg_kernel(dst32, ew)
    dp = degp[:, :N, 0].T
    Y, dis = _y_dis(x, W, dp)

    A = _edge_kernel(src32, dst32, ew, Y)

    return _final(A[0, :N], A[1, :N], Y, dis, W_lin, b_lin.reshape(1, 1))


# speedup vs baseline: 23.7581x; 2.0986x over previous
import functools

import jax
import jax.numpy as jnp
from jax import lax
from jax.experimental import pallas as pl
from jax.experimental.pallas import tpu as pltpu
from jax.experimental.pallas import tpu_sc as plsc

N = 10000
E = 320000
C = 128

_NC = 2
_NS = 16
_NW = _NC * _NS
_H = C // _NC
_K = 80
_NP = 10240
_RPT = _NP // _NS

_EPT_D = E // _NW
_NCH_D = _EPT_D // _K
_EPT_E = E // _NW
_NCH_E = _EPT_E // _K

_mesh = plsc.VectorSubcoreMesh(core_axis_name="c", subcore_axis_name="s")


def _score_body(x_ref, p_ref, o_ref):
    pv = p_ref[...]
    inv = lax.rsqrt(jnp.sum(pv * pv))
    z = jnp.sum(x_ref[...] * pv, axis=1, keepdims=True)
    o_ref[...] = jnp.tanh(z * inv)


def _scores(x, p2):
    return pl.pallas_call(
        _score_body,
        grid=(10,),
        in_specs=[
            pl.BlockSpec((N // 10, C), lambda i: (i, 0)),
            pl.BlockSpec((1, C), lambda i: (0, 0)),
        ],
        out_specs=pl.BlockSpec((N // 10, 1), lambda i: (i, 0)),
        out_shape=jax.ShapeDtypeStruct((N, 1), jnp.float32),
    )(x, p2)


def _gru_body(xt_ref, wi_ref, wh_ref, bi_ref, bh_ref, h0_ref, w_ref):
    xt = xt_ref[...]
    h0 = h0_ref[...]
    dn = (((1,), (1,)), ((), ()))
    gi = lax.dot_general(xt, wi_ref[...], dn,
                         preferred_element_type=jnp.float32) + bi_ref[...]
    gh = lax.dot_general(h0, wh_ref[...], dn,
                         preferred_element_type=jnp.float32) + bh_ref[...]
    r = jax.nn.sigmoid(gi[:, :C] + gh[:, :C])
    z = jax.nn.sigmoid(gi[:, C:2 * C] + gh[:, C:2 * C])
    nc = jnp.tanh(gi[:, 2 * C:] + r * gh[:, 2 * C:])
    w_ref[...] = (1.0 - z) * nc + z * h0


def _gru(x_tilde, W_ih, W_hh, bi2, bh2, W_init):
    return pl.pallas_call(
        _gru_body,
        out_shape=jax.ShapeDtypeStruct((C, C), jnp.float32),
    )(x_tilde, W_ih, W_hh, bi2, bh2, W_init)


@functools.partial(
    pl.kernel,
    out_type=jax.ShapeDtypeStruct((_NC, _NP, 16), jnp.float32),
    mesh=_mesh,
    scratch_types=[
        pltpu.VMEM_SHARED((_NP, 16), jnp.float32),
        pltpu.VMEM((_EPT_D,), jnp.float32),
        pltpu.VMEM((_K,), jnp.int32),
        pltpu.VMEM((_K,), jnp.int32),
        pltpu.VMEM((_K, 16), jnp.float32),
        pltpu.VMEM((_RPT // 4, 16), jnp.float32),
        pltpu.SemaphoreType.DMA,
        pltpu.SemaphoreType.DMA,
    ],
)
def _deg_kernel(dst_hbm, w_hbm, out_hbm, dacc, wall, didx0, didx1,
                wrow, zbuf, sem0, sem1):
    c = lax.axis_index("c")
    s = lax.axis_index("s")
    wid = c * _NS + s
    base = wid * _EPT_D

    def _zrow(i, _):
        zbuf[i, :] = jnp.zeros((16,), jnp.float32)
        return 0
    lax.fori_loop(0, _RPT // 4, _zrow, 0)
    for j in range(4):
        pltpu.sync_copy(zbuf, dacc.at[pl.ds(s * _RPT + j * (_RPT // 4),
                                            _RPT // 4), :])
    pltpu.sync_copy(w_hbm.at[pl.ds(base, _EPT_D)], wall)
    plsc.subcore_barrier()

    def _issue(j, didx_b, sem_b):
        pltpu.async_copy(dst_hbm.at[pl.ds(base + j * _K, _K)], didx_b, sem_b)

    def _wait(didx_b, sem_b):
        pltpu.make_async_copy(dst_hbm.at[pl.ds(0, _K)], didx_b, sem_b).wait()

    def _proc(j, didx_b):
        def _grp(g, __):
            wg = wall[pl.ds(j * _K + g * 16, 16)]
            for l in range(16):
                wrow[g * 16 + l, :] = jnp.broadcast_to(wg[l], (16,))
            return 0
        lax.fori_loop(0, _K // 16, _grp, 0)
        pltpu.sync_copy(wrow, dacc.at[didx_b], add=True)

    _issue(0, didx0, sem0)

    def _pair(i, _):
        j0 = 2 * i
        _issue(j0 + 1, didx1, sem1)
        _wait(didx0, sem0)
        _proc(j0, didx0)
        _issue(j0 + 2, didx0, sem0)
        _wait(didx1, sem1)
        _proc(j0 + 1, didx1)
        return 0
    lax.fori_loop(0, (_NCH_D - 1) // 2, _pair, 0)
    _wait(didx0, sem0)
    _proc(_NCH_D - 1, didx0)

    plsc.subcore_barrier()
    pltpu.sync_copy(dacc.at[pl.ds(s * _RPT, _RPT), :],
                    out_hbm.at[c, pl.ds(s * _RPT, _RPT), :])


def _y_body(x_ref, w_ref, dp_ref, y_ref, dis_ref):
    d = jnp.sum(dp_ref[...], axis=1, keepdims=True) + 1.0
    dis = lax.rsqrt(d)
    xw = lax.dot_general(x_ref[...], w_ref[...], (((1,), (0,)), ((), ())),
                         preferred_element_type=jnp.float32)
    y_ref[...] = xw * dis
    dis_ref[...] = dis


def _y_dis(xp, W, dp):
    return pl.pallas_call(
        _y_body,
        grid=(16,),
        in_specs=[
            pl.BlockSpec((_RPT, C), lambda i: (i, 0)),
            pl.BlockSpec((C, C), lambda i: (0, 0)),
            pl.BlockSpec((_RPT, _NC), lambda i: (i, 0)),
        ],
        out_specs=[
            pl.BlockSpec((_RPT, C), lambda i: (i, 0)),
            pl.BlockSpec((_RPT, 1), lambda i: (i, 0)),
        ],
        out_shape=[
            jax.ShapeDtypeStruct((_NP, C), jnp.float32),
            jax.ShapeDtypeStruct((_NP, 1), jnp.float32),
        ],
    )(xp, W, dp)


@functools.partial(
    pl.kernel,
    out_type=jax.ShapeDtypeStruct((_NC, _NP, C), jnp.float32),
    mesh=_mesh,
    scratch_types=[
        pltpu.VMEM_SHARED((_NP, C), jnp.float32),
        pltpu.VMEM((_EPT_E,), jnp.int32),
        pltpu.VMEM((_EPT_E,), jnp.float32),
        pltpu.VMEM((_K,), jnp.int32),
        pltpu.VMEM((_K,), jnp.int32),
        pltpu.VMEM((_K, C), jnp.float32),
        pltpu.VMEM((_K, C), jnp.float32),
        pltpu.VMEM((_RPT // 16, C), jnp.float32),
        pltpu.SemaphoreType.DMA,
        pltpu.SemaphoreType.DMA,
    ],
)
def _edge_kernel(src_hbm, dst_hbm, w_hbm, y_hbm, out_hbm,
                 acc, srcv, wall, didx0, didx1, rows0, rows1,
                 zbuf, sem0, sem1):
    c = lax.axis_index("c")
    s = lax.axis_index("s")
    base = (c * _NS + s) * _EPT_E

    def _zrow(i, _):
        for ci in range(C // 16):
            zbuf[i, pl.ds(ci * 16, 16)] = jnp.zeros((16,), jnp.float32)
        return 0
    lax.fori_loop(0, _RPT // 16, _zrow, 0)
    for j in range(16):
        pltpu.sync_copy(zbuf, acc.at[pl.ds(s * _RPT + j * (_RPT // 16),
                                           _RPT // 16), :])
    pltpu.sync_copy(src_hbm.at[pl.ds(base, _EPT_E)], srcv)
    pltpu.sync_copy(w_hbm.at[pl.ds(base, _EPT_E)], wall)
    plsc.subcore_barrier()

    def _issue(j, didx_b, rows_b, sem_b):
        pltpu.async_copy(dst_hbm.at[pl.ds(base + j * _K, _K)], didx_b, sem_b)
        pltpu.async_copy(y_hbm.at[srcv.at[pl.ds(j * _K, _K)]], rows_b, sem_b)

    def _wait(didx_b, rows_b, sem_b):
        pltpu.make_async_copy(dst_hbm.at[pl.ds(0, _K)], didx_b, sem_b).wait()
        pltpu.make_async_copy(y_hbm.at[pl.ds(0, _K), :], rows_b,
                              sem_b).wait()

    def _proc(j, didx_b, rows_b):
        def _grp(g, __):
            wg = wall[pl.ds(j * _K + g * 16, 16)]
            for l in range(16):
                wj = wg[l]
                r = g * 16 + l
                for ci in range(C // 16):
                    sl = pl.ds(ci * 16, 16)
                    rows_b[r, sl] = rows_b[r, sl] * wj
            return 0
        lax.fori_loop(0, _K // 16, _grp, 0)
        pltpu.sync_copy(rows_b, acc.at[didx_b], add=True)

    _issue(0, didx0, rows0, sem0)

    def _pair(i, _):
        j0 = 2 * i
        _issue(j0 + 1, didx1, rows1, sem1)
        _wait(didx0, rows0, sem0)
        _proc(j0, didx0, rows0)
        _issue(j0 + 2, didx0, rows0, sem0)
        _wait(didx1, rows1, sem1)
        _proc(j0 + 1, didx1, rows1)
        return 0
    lax.fori_loop(0, (_NCH_E - 1) // 2, _pair, 0)
    _wait(didx0, rows0, sem0)
    _proc(_NCH_E - 1, didx0, rows0)

    plsc.subcore_barrier()
    pltpu.sync_copy(acc.at[pl.ds(s * _RPT, _RPT), :],
                    out_hbm.at[c, pl.ds(s * _RPT, _RPT), :])


def _fin_body(a0_ref, a1_ref, y_ref, dis_ref, wl_ref, bl_ref, o_ref):
    h = jnp.maximum((a0_ref[...] + a1_ref[...] + y_ref[...]) * dis_ref[...],
                    0.0)
    o_ref[...] = lax.dot_general(h, wl_ref[...], (((1,), (0,)), ((), ())),
                                 preferred_element_type=jnp.float32) + bl_ref[...]


def _final(a0, a1, y, dis, W_lin, bl2):
    blk = N // 10
    return pl.pallas_call(
        _fin_body,
        grid=(10,),
        in_specs=[
            pl.BlockSpec((blk, C), lambda i: (i, 0)),
            pl.BlockSpec((blk, C), lambda i: (i, 0)),
            pl.BlockSpec((blk, C), lambda i: (i, 0)),
            pl.BlockSpec((blk, 1), lambda i: (i, 0)),
            pl.BlockSpec((C, 1), lambda i: (0, 0)),
            pl.BlockSpec((1, 1), lambda i: (0, 0)),
        ],
        out_specs=pl.BlockSpec((blk, 1), lambda i: (i, 0)),
        out_shape=jax.ShapeDtypeStruct((N, 1), jnp.float32),
    )(a0, a1, y, dis, W_lin, bl2)


def kernel(x, edge_index, edge_weight, p, W_ih, W_hh, b_ih, b_hh,
           W_init, W_lin, b_lin):
    src32 = edge_index[0].astype(jnp.int32)
    dst32 = edge_index[1].astype(jnp.int32)
    ew = edge_weight.astype(jnp.float32)

    score = _scores(x, p.reshape(1, C))[:, 0]
    topv, perm = lax.top_k(score, C)
    x_tilde = x[perm] * topv[:, None]

    W = _gru(x_tilde, W_ih, W_hh, b_ih.reshape(1, 3 * C),
             b_hh.reshape(1, 3 * C), W_init)

    degp = _deg_kernel(dst32, ew)
    dp = degp[:, :, 0].T
    xp = jnp.concatenate([x, jnp.zeros((_NP - N, C), jnp.float32)], axis=0)
    Y, dis = _y_dis(xp, W, dp)

    A = _edge_kernel(src32, dst32, ew, Y)

    return _final(A[0, :N], A[1, :N], Y[:N], dis[:N],
                  W_lin, b_lin.reshape(1, 1))


# speedup vs baseline: 24.0712x; 1.0132x over previous
import functools

import jax
import jax.numpy as jnp
from jax import lax
from jax.experimental import pallas as pl
from jax.experimental.pallas import tpu as pltpu
from jax.experimental.pallas import tpu_sc as plsc

N = 10000
E = 320000
C = 128

_NC = 2
_NS = 16
_NW = _NC * _NS
_H = C // _NC
_K = 80
_NP = 10240
_RPT = _NP // _NS

_EPT_D = E // _NW
_NCH_D = _EPT_D // _K
_EPT_E = E // _NW
_NCH_E = _EPT_E // _K

_mesh = plsc.VectorSubcoreMesh(core_axis_name="c", subcore_axis_name="s")


def _score_body(x_ref, p_ref, o_ref):
    pv = p_ref[...]
    inv = lax.rsqrt(jnp.sum(pv * pv))
    z = jnp.sum(x_ref[...] * pv, axis=1, keepdims=True)
    o_ref[...] = jnp.tanh(z * inv)


def _scores(x, p2):
    return pl.pallas_call(
        _score_body,
        grid=(10,),
        in_specs=[
            pl.BlockSpec((N // 10, C), lambda i: (i, 0)),
            pl.BlockSpec((1, C), lambda i: (0, 0)),
        ],
        out_specs=pl.BlockSpec((N // 10, 1), lambda i: (i, 0)),
        out_shape=jax.ShapeDtypeStruct((N, 1), jnp.float32),
    )(x, p2)


def _gru_body(xt_ref, wi_ref, wh_ref, bi_ref, bh_ref, h0_ref, w_ref):
    xt = xt_ref[...]
    h0 = h0_ref[...]
    dn = (((1,), (1,)), ((), ()))
    gi = lax.dot_general(xt, wi_ref[...], dn,
                         preferred_element_type=jnp.float32) + bi_ref[...]
    gh = lax.dot_general(h0, wh_ref[...], dn,
                         preferred_element_type=jnp.float32) + bh_ref[...]
    r = jax.nn.sigmoid(gi[:, :C] + gh[:, :C])
    z = jax.nn.sigmoid(gi[:, C:2 * C] + gh[:, C:2 * C])
    nc = jnp.tanh(gi[:, 2 * C:] + r * gh[:, 2 * C:])
    w_ref[...] = (1.0 - z) * nc + z * h0


def _gru(x_tilde, W_ih, W_hh, bi2, bh2, W_init):
    return pl.pallas_call(
        _gru_body,
        out_shape=jax.ShapeDtypeStruct((C, C), jnp.float32),
    )(x_tilde, W_ih, W_hh, bi2, bh2, W_init)


@functools.partial(
    pl.kernel,
    out_type=jax.ShapeDtypeStruct((_NC, _NP, 16), jnp.float32),
    mesh=_mesh,
    scratch_types=[
        pltpu.VMEM_SHARED((_NP, 16), jnp.float32),
        pltpu.VMEM((_EPT_D,), jnp.float32),
        pltpu.VMEM((_K,), jnp.int32),
        pltpu.VMEM((_K,), jnp.int32),
        pltpu.VMEM((_K, 16), jnp.float32),
        pltpu.VMEM((_RPT // 4, 16), jnp.float32),
        pltpu.SemaphoreType.DMA,
        pltpu.SemaphoreType.DMA,
    ],
)
def _deg_kernel(dst_hbm, w_hbm, out_hbm, dacc, wall, didx0, didx1,
                wrow, zbuf, sem0, sem1):
    c = lax.axis_index("c")
    s = lax.axis_index("s")
    wid = c * _NS + s
    base = wid * _EPT_D

    def _zrow(i, _):
        zbuf[i, :] = jnp.zeros((16,), jnp.float32)
        return 0
    lax.fori_loop(0, _RPT // 4, _zrow, 0)
    for j in range(4):
        pltpu.sync_copy(zbuf, dacc.at[pl.ds(s * _RPT + j * (_RPT // 4),
                                            _RPT // 4), :])
    pltpu.sync_copy(w_hbm.at[pl.ds(base, _EPT_D)], wall)
    plsc.subcore_barrier()

    def _issue(j, didx_b, sem_b):
        pltpu.async_copy(dst_hbm.at[pl.ds(base + j * _K, _K)], didx_b, sem_b)

    def _wait(didx_b, sem_b):
        pltpu.make_async_copy(dst_hbm.at[pl.ds(0, _K)], didx_b, sem_b).wait()

    def _proc(j, didx_b):
        def _grp(g, __):
            wg = wall[pl.ds(j * _K + g * 16, 16)]
            for l in range(16):
                wrow[g * 16 + l, :] = jnp.broadcast_to(wg[l], (16,))
            return 0
        lax.fori_loop(0, _K // 16, _grp, 0)
        pltpu.sync_copy(wrow, dacc.at[didx_b], add=True)

    _issue(0, didx0, sem0)

    def _pair(i, _):
        j0 = 2 * i
        _issue(j0 + 1, didx1, sem1)
        _wait(didx0, sem0)
        _proc(j0, didx0)
        _issue(j0 + 2, didx0, sem0)
        _wait(didx1, sem1)
        _proc(j0 + 1, didx1)
        return 0
    lax.fori_loop(0, (_NCH_D - 1) // 2, _pair, 0)
    _wait(didx0, sem0)
    _proc(_NCH_D - 1, didx0)

    plsc.subcore_barrier()
    pltpu.sync_copy(dacc.at[pl.ds(s * _RPT, _RPT), :],
                    out_hbm.at[c, pl.ds(s * _RPT, _RPT), :])


def _y_body(x_ref, w_ref, dp_ref, y_ref, dis_ref):
    d = jnp.sum(dp_ref[...], axis=1, keepdims=True) + 1.0
    dis = lax.rsqrt(d)
    xw = lax.dot_general(x_ref[...], w_ref[...], (((1,), (0,)), ((), ())),
                         preferred_element_type=jnp.float32)
    y_ref[...] = xw * dis
    dis_ref[...] = dis


def _y_dis(xp, W, dp):
    return pl.pallas_call(
        _y_body,
        grid=(16,),
        in_specs=[
            pl.BlockSpec((_RPT, C), lambda i: (i, 0)),
            pl.BlockSpec((C, C), lambda i: (0, 0)),
            pl.BlockSpec((_RPT, _NC), lambda i: (i, 0)),
        ],
        out_specs=[
            pl.BlockSpec((_RPT, C), lambda i: (i, 0)),
            pl.BlockSpec((_RPT, 1), lambda i: (i, 0)),
        ],
        out_shape=[
            jax.ShapeDtypeStruct((_NP, C), jnp.float32),
            jax.ShapeDtypeStruct((_NP, 1), jnp.float32),
        ],
    )(xp, W, dp)


@functools.partial(
    pl.kernel,
    out_type=jax.ShapeDtypeStruct((_NC, _NP, C), jnp.float32),
    mesh=_mesh,
    scratch_types=[
        pltpu.VMEM_SHARED((_NP, C), jnp.float32),
        pltpu.VMEM((_EPT_E,), jnp.int32),
        pltpu.VMEM((_EPT_E,), jnp.float32),
        pltpu.VMEM((_K,), jnp.int32),
        pltpu.VMEM((_K,), jnp.int32),
        pltpu.VMEM((_K, C), jnp.float32),
        pltpu.VMEM((_K, C), jnp.float32),
        pltpu.VMEM((_RPT // 16, C), jnp.float32),
        pltpu.SemaphoreType.DMA,
        pltpu.SemaphoreType.DMA,
    ],
)
def _edge_kernel(src_hbm, dst_hbm, w_hbm, y_hbm, out_hbm,
                 acc, srcv, wall, didx0, didx1, rows0, rows1,
                 zbuf, sem0, sem1):
    c = lax.axis_index("c")
    s = lax.axis_index("s")
    base = (c * _NS + s) * _EPT_E

    def _zrow(i, _):
        for ci in range(C // 16):
            zbuf[i, pl.ds(ci * 16, 16)] = jnp.zeros((16,), jnp.float32)
        return 0
    lax.fori_loop(0, _RPT // 16, _zrow, 0)
    for j in range(16):
        pltpu.sync_copy(zbuf, acc.at[pl.ds(s * _RPT + j * (_RPT // 16),
                                           _RPT // 16), :])
    pltpu.sync_copy(src_hbm.at[pl.ds(base, _EPT_E)], srcv)
    pltpu.sync_copy(w_hbm.at[pl.ds(base, _EPT_E)], wall)
    plsc.subcore_barrier()

    def _issue(j, didx_b, rows_b, sem_b):
        pltpu.async_copy(dst_hbm.at[pl.ds(base + j * _K, _K)], didx_b, sem_b)
        pltpu.async_copy(y_hbm.at[srcv.at[pl.ds(j * _K, _K)]], rows_b, sem_b)

    def _wait(didx_b, rows_b, sem_b):
        pltpu.make_async_copy(dst_hbm.at[pl.ds(0, _K)], didx_b, sem_b).wait()
        pltpu.make_async_copy(y_hbm.at[pl.ds(0, _K), :], rows_b,
                              sem_b).wait()

    def _proc(j, didx_b, rows_b):
        def _grp(g, __):
            wg = wall[pl.ds(j * _K + g * 16, 16)]
            for l in range(16):
                wj = wg[l]
                r = g * 16 + l
                for ci in range(C // 16):
                    sl = pl.ds(ci * 16, 16)
                    rows_b[r, sl] = rows_b[r, sl] * wj
            return 0
        lax.fori_loop(0, _K // 16, _grp, 0)
        pltpu.sync_copy(rows_b, acc.at[didx_b], add=True)

    _issue(0, didx0, rows0, sem0)

    def _pair(i, _):
        j0 = 2 * i
        _issue(j0 + 1, didx1, rows1, sem1)
        _wait(didx0, rows0, sem0)
        _proc(j0, didx0, rows0)
        _issue(j0 + 2, didx0, rows0, sem0)
        _wait(didx1, rows1, sem1)
        _proc(j0 + 1, didx1, rows1)
        return 0
    lax.fori_loop(0, (_NCH_E - 1) // 2, _pair, 0)
    _wait(didx0, rows0, sem0)
    _proc(_NCH_E - 1, didx0, rows0)

    plsc.subcore_barrier()
    pltpu.sync_copy(acc.at[pl.ds(s * _RPT, _RPT), :],
                    out_hbm.at[c, pl.ds(s * _RPT, _RPT), :])


def _fin_body(a0_ref, a1_ref, y_ref, dis_ref, wl_ref, bl_ref, o_ref):
    h = jnp.maximum((a0_ref[0] + a1_ref[0] + y_ref[...]) * dis_ref[...],
                    0.0)
    o_ref[...] = lax.dot_general(h, wl_ref[...], (((1,), (0,)), ((), ())),
                                 preferred_element_type=jnp.float32) + bl_ref[...]


def _final(a, y, dis, W_lin, bl2):
    return pl.pallas_call(
        _fin_body,
        grid=(16,),
        in_specs=[
            pl.BlockSpec((1, _RPT, C), lambda i: (0, i, 0)),
            pl.BlockSpec((1, _RPT, C), lambda i: (1, i, 0)),
            pl.BlockSpec((_RPT, C), lambda i: (i, 0)),
            pl.BlockSpec((_RPT, 1), lambda i: (i, 0)),
            pl.BlockSpec((C, 1), lambda i: (0, 0)),
            pl.BlockSpec((1, 1), lambda i: (0, 0)),
        ],
        out_specs=pl.BlockSpec((_RPT, 1), lambda i: (i, 0)),
        out_shape=jax.ShapeDtypeStruct((_NP, 1), jnp.float32),
    )(a, a, y, dis, W_lin, bl2)


def kernel(x, edge_index, edge_weight, p, W_ih, W_hh, b_ih, b_hh,
           W_init, W_lin, b_lin):
    src32 = edge_index[0].astype(jnp.int32)
    dst32 = edge_index[1].astype(jnp.int32)
    ew = edge_weight.astype(jnp.float32)

    score = _scores(x, p.reshape(1, C))[:, 0]
    topv, perm = lax.top_k(score, C)
    x_tilde = x[perm] * topv[:, None]

    W = _gru(x_tilde, W_ih, W_hh, b_ih.reshape(1, 3 * C),
             b_hh.reshape(1, 3 * C), W_init)

    degp = _deg_kernel(dst32, ew)
    dp = degp[:, :, 0].T
    Y, dis = _y_dis(x, W, dp)

    A = _edge_kernel(src32, dst32, ew, Y)

    return _final(A, Y, dis, W_lin, b_lin.reshape(1, 1))[:N]
